# baseline (speedup 1.0000x reference)
import functools

import jax
import jax.numpy as jnp
from jax import lax
from jax.experimental import pallas as pl
from jax.experimental.pallas import tpu as pltpu
from jax.experimental.pallas import tpu_sc as plsc

B = 4096
D = 128
NEG = 5
EPS = 1e-8


def _sc_neg_stats(batch_users, neg_idx_flat, total_user, total_sub):
    info = plsc.get_sparse_core_info()
    NC, NS, L = info.num_cores, info.num_subcores, info.num_lanes
    NW = NC * NS
    bpw = B // NW
    vpw = bpw * NEG
    K = D // L

    mesh = plsc.VectorSubcoreMesh(core_axis_name="c", subcore_axis_name="s")

    @functools.partial(
        pl.kernel,
        mesh=mesh,
        compiler_params=pltpu.CompilerParams(needs_layout_passes=False,
                                             use_tc_tiling_on_sc=False),
        out_type=[
            jax.ShapeDtypeStruct((B * NEG, L), jnp.float32),
            jax.ShapeDtypeStruct((B * NEG, L), jnp.float32),
            jax.ShapeDtypeStruct((B * NEG, L), jnp.float32),
        ],
        scratch_types=[
            pltpu.VMEM((bpw,), jnp.int32),
            pltpu.VMEM((vpw,), jnp.int32),
            pltpu.VMEM((bpw // 8, D), jnp.float32),
            pltpu.VMEM((bpw // 8, D), jnp.float32),
            pltpu.VMEM((vpw // 8, D), jnp.float32),
            pltpu.VMEM((vpw // 8, D), jnp.float32),
            pltpu.VMEM((vpw, L), jnp.float32),
            pltpu.VMEM((vpw, L), jnp.float32),
            pltpu.VMEM((vpw, L), jnp.float32),
            pltpu.SemaphoreType.DMA,
            pltpu.SemaphoreType.DMA,
        ],
    )
    def k(bu_hbm, ni_hbm, tu_hbm, ts_hbm, dot_out, nv2_out, nu2_out,
          uidx_v, nidx_v, u_buf0, u_buf1, v_buf0, v_buf1,
          rdot, rnv2, rnu2, sem0, sem1):
        u_bufs = (u_buf0, u_buf1)
        v_bufs = (v_buf0, v_buf1)
        wid = lax.axis_index("s") * NC + lax.axis_index("c")
        base = wid * bpw
        vbase = wid * vpw
        P = 8
        rp = bpw // P
        vp = vpw // P
        sems = (sem0, sem1)

        pltpu.sync_copy(bu_hbm.at[pl.ds(base, bpw)], uidx_v)
        pltpu.sync_copy(ni_hbm.at[pl.ds(vbase, vpw)], nidx_v)

        def fire(p):
            b = p % 2
            return [
                pltpu.async_copy(tu_hbm.at[uidx_v.at[pl.ds(p * rp, rp)]],
                                 u_bufs[b], sems[b]),
                pltpu.async_copy(ts_hbm.at[nidx_v.at[pl.ds(p * vp, vp)]],
                                 v_bufs[b], sems[b]),
            ]

        pending = {0: fire(0)}
        for p in range(P):
            b = p % 2
            for cp in pending.pop(p):
                cp.wait()
            if p + 1 < P:
                pending[p + 1] = fire(p + 1)

            def row_body(r, b=b, p=p):
                ub, vb = u_bufs[b], v_bufs[b]
                u = [ub[r, pl.ds(kk * L, L)] for kk in range(K)]
                accu = u[0] * u[0]
                for kk in range(1, K):
                    accu = accu + u[kk] * u[kk]
                lr0 = r * NEG
                gr0 = p * vp + r * NEG
                for j in range(NEG):
                    v0 = vb[lr0 + j, pl.ds(0, L)]
                    accd = u[0] * v0
                    accv = v0 * v0
                    for kk in range(1, K):
                        vk = vb[lr0 + j, pl.ds(kk * L, L)]
                        accd = accd + u[kk] * vk
                        accv = accv + vk * vk
                    rdot[gr0 + j, :] = accd
                    rnv2[gr0 + j, :] = accv
                    rnu2[gr0 + j, :] = accu

            plsc.parallel_loop(0, rp, 1, unroll=2)(row_body)

        pltpu.sync_copy(rdot, dot_out.at[pl.ds(vbase, vpw)])
        pltpu.sync_copy(rnv2, nv2_out.at[pl.ds(vbase, vpw)])
        pltpu.sync_copy(rnu2, nu2_out.at[pl.ds(vbase, vpw)])

    return k(batch_users, neg_idx_flat, total_user, total_sub)


def _tc_positive(ue, se):
    def body(ue_ref, se_ref, out_ref):
        u = ue_ref[...]
        s = se_ref[...]
        dot = jnp.sum(u * s, axis=1)
        na = jnp.sqrt(jnp.sum(u * u, axis=1))
        nb = jnp.sqrt(jnp.sum(s * s, axis=1))
        out_ref[0, 0] = jnp.sum(dot / jnp.maximum(na * nb, EPS))

    return pl.pallas_call(
        body,
        out_shape=jax.ShapeDtypeStruct((1, 1), jnp.float32),
        out_specs=pl.BlockSpec(memory_space=pltpu.SMEM),
    )(ue, se)


def _tc_finish(pos_sum, dotp, nv2p, nu2p):
    def body(pos_ref, dotp_ref, nv2p_ref, nu2p_ref, out_ref):
        ri = lax.broadcasted_iota(jnp.int32, (128, 8), 0)
        ci = lax.broadcasted_iota(jnp.int32, (128, 8), 1)
        m = (ri // 16 == ci).astype(jnp.float32)
        d = jnp.dot(dotp_ref[...], m, preferred_element_type=jnp.float32)
        v2 = jnp.dot(nv2p_ref[...], m, preferred_element_type=jnp.float32)
        u2 = jnp.dot(nu2p_ref[...], m, preferred_element_type=jnp.float32)
        c = d / jnp.maximum(jnp.sqrt(u2) * jnp.sqrt(v2), EPS)
        neg_sum = jnp.sum(c)

        out_ref[0, 0] = 1.0 - pos_ref[0, 0] / B + neg_sum / (NEG * B)

    return pl.pallas_call(
        body,
        out_shape=jax.ShapeDtypeStruct((1, 1), jnp.float32),
        out_specs=pl.BlockSpec(memory_space=pltpu.SMEM),
    )(pos_sum, dotp, nv2p, nu2p)


def kernel(user_embeddings, subreddit_embeddings, batch_users,
           batch_subreddits, total_user_embeddings, total_subreddit_embeddings,
           negative_indices):
    del batch_subreddits
    bu = batch_users.astype(jnp.int32)
    ni = negative_indices.astype(jnp.int32).reshape(B * NEG)
    dotp, nv2p, nu2p = _sc_neg_stats(bu, ni, total_user_embeddings,
                                     total_subreddit_embeddings)
    pos = _tc_positive(user_embeddings, subreddit_embeddings)
    r = B * NEG * 16 // 128
    loss = _tc_finish(pos, dotp.reshape(r, 128), nv2p.reshape(r, 128),
                      nu2p.reshape(r, 128))
    return loss[0, 0]


# speedup vs baseline: 1.0880x; 1.0880x over previous
import functools

import jax
import jax.numpy as jnp
from jax import lax
from jax.experimental import pallas as pl
from jax.experimental.pallas import tpu as pltpu
from jax.experimental.pallas import tpu_sc as plsc

B = 4096
D = 128
NEG = 5
EPS = 1e-8


def _sc_neg_stats(batch_users, neg_idx_flat, total_user, total_sub):
    info = plsc.get_sparse_core_info()
    NC, NS, L = info.num_cores, info.num_subcores, info.num_lanes
    NW = NC * NS
    bpw = B // NW
    vpw = bpw * NEG
    K = D // L

    mesh = plsc.VectorSubcoreMesh(core_axis_name="c", subcore_axis_name="s")

    @functools.partial(
        pl.kernel,
        mesh=mesh,
        compiler_params=pltpu.CompilerParams(needs_layout_passes=False,
                                             use_tc_tiling_on_sc=False),
        out_type=[
            jax.ShapeDtypeStruct((B * NEG, L), jnp.float32),
            jax.ShapeDtypeStruct((B * NEG, L), jnp.float32),
            jax.ShapeDtypeStruct((B * NEG, L), jnp.float32),
        ],
        scratch_types=[
            pltpu.VMEM((bpw,), jnp.int32),
            pltpu.VMEM((vpw,), jnp.int32),
            pltpu.VMEM((bpw // 8, D), jnp.float32),
            pltpu.VMEM((bpw // 8, D), jnp.float32),
            pltpu.VMEM((bpw // 8, D), jnp.float32),
            pltpu.VMEM((vpw // 8, D), jnp.float32),
            pltpu.VMEM((vpw // 8, D), jnp.float32),
            pltpu.VMEM((vpw // 8, D), jnp.float32),
            pltpu.VMEM((vpw, L), jnp.float32),
            pltpu.VMEM((vpw, L), jnp.float32),
            pltpu.VMEM((vpw, L), jnp.float32),
            pltpu.SemaphoreType.DMA,
            pltpu.SemaphoreType.DMA,
            pltpu.SemaphoreType.DMA,
        ],
    )
    def k(bu_hbm, ni_hbm, tu_hbm, ts_hbm, dot_out, nv2_out, nu2_out,
          uidx_v, nidx_v, u_buf0, u_buf1, u_buf2, v_buf0, v_buf1, v_buf2,
          rdot, rnv2, rnu2, sem0, sem1, sem2):
        u_bufs = (u_buf0, u_buf1, u_buf2)
        v_bufs = (v_buf0, v_buf1, v_buf2)
        wid = lax.axis_index("s") * NC + lax.axis_index("c")
        base = wid * bpw
        vbase = wid * vpw
        P = 8
        rp = bpw // P
        vp = vpw // P
        sems = (sem0, sem1, sem2)
        NB = 3

        pltpu.sync_copy(bu_hbm.at[pl.ds(base, bpw)], uidx_v)
        pltpu.sync_copy(ni_hbm.at[pl.ds(vbase, vpw)], nidx_v)

        def fire(p):
            b = p % NB
            return [
                pltpu.async_copy(tu_hbm.at[uidx_v.at[pl.ds(p * rp, rp)]],
                                 u_bufs[b], sems[b]),
                pltpu.async_copy(ts_hbm.at[nidx_v.at[pl.ds(p * vp, vp)]],
                                 v_bufs[b], sems[b]),
            ]

        pending = {0: fire(0), 1: fire(1)}
        for p in range(P):
            b = p % NB
            for cp in pending.pop(p):
                cp.wait()
            nxt = p + NB - 1
            if nxt < P:
                pending[nxt] = fire(nxt)

            def row_body(r, carry, b=b, p=p):
                ub, vb = u_bufs[b], v_bufs[b]
                u = [ub[r, pl.ds(kk * L, L)] for kk in range(K)]
                accu = u[0] * u[0]
                for kk in range(1, K):
                    accu = accu + u[kk] * u[kk]
                lr0 = r * NEG
                gr0 = p * vp + r * NEG
                for j in range(NEG):
                    v0 = vb[lr0 + j, pl.ds(0, L)]
                    accd = u[0] * v0
                    accv = v0 * v0
                    for kk in range(1, K):
                        vk = vb[lr0 + j, pl.ds(kk * L, L)]
                        accd = accd + u[kk] * vk
                        accv = accv + vk * vk
                    rdot[gr0 + j, :] = accd
                    rnv2[gr0 + j, :] = accv
                    rnu2[gr0 + j, :] = accu
                return carry

            lax.fori_loop(0, rp, row_body, 0)

        pltpu.sync_copy(rdot, dot_out.at[pl.ds(vbase, vpw)])
        pltpu.sync_copy(rnv2, nv2_out.at[pl.ds(vbase, vpw)])
        pltpu.sync_copy(rnu2, nu2_out.at[pl.ds(vbase, vpw)])

    return k(batch_users, neg_idx_flat, total_user, total_sub)


def _tc_positive(ue, se):
    def body(ue_ref, se_ref, out_ref):
        u = ue_ref[...]
        s = se_ref[...]
        dot = jnp.sum(u * s, axis=1)
        na = jnp.sqrt(jnp.sum(u * u, axis=1))
        nb = jnp.sqrt(jnp.sum(s * s, axis=1))
        out_ref[0, 0] = jnp.sum(dot / jnp.maximum(na * nb, EPS))

    return pl.pallas_call(
        body,
        out_shape=jax.ShapeDtypeStruct((1, 1), jnp.float32),
        out_specs=pl.BlockSpec(memory_space=pltpu.SMEM),
    )(ue, se)


def _tc_finish(pos_sum, dotp, nv2p, nu2p):
    def body(pos_ref, dotp_ref, nv2p_ref, nu2p_ref, out_ref):
        ri = lax.broadcasted_iota(jnp.int32, (128, 8), 0)
        ci = lax.broadcasted_iota(jnp.int32, (128, 8), 1)
        m = (ri // 16 == ci).astype(jnp.float32)
        d = jnp.dot(dotp_ref[...], m, preferred_element_type=jnp.float32)
        v2 = jnp.dot(nv2p_ref[...], m, preferred_element_type=jnp.float32)
        u2 = jnp.dot(nu2p_ref[...], m, preferred_element_type=jnp.float32)
        c = d / jnp.maximum(jnp.sqrt(u2) * jnp.sqrt(v2), EPS)
        neg_sum = jnp.sum(c)

        out_ref[0, 0] = 1.0 - pos_ref[0, 0] / B + neg_sum / (NEG * B)

    return pl.pallas_call(
        body,
        out_shape=jax.ShapeDtypeStruct((1, 1), jnp.float32),
        out_specs=pl.BlockSpec(memory_space=pltpu.SMEM),
    )(pos_sum, dotp, nv2p, nu2p)


def kernel(user_embeddings, subreddit_embeddings, batch_users,
           batch_subreddits, total_user_embeddings, total_subreddit_embeddings,
           negative_indices):
    del batch_subreddits
    bu = batch_users.astype(jnp.int32)
    ni = negative_indices.astype(jnp.int32).reshape(B * NEG)
    dotp, nv2p, nu2p = _sc_neg_stats(bu, ni, total_user_embeddings,
                                     total_subreddit_embeddings)
    pos = _tc_positive(user_embeddings, subreddit_embeddings)
    r = B * NEG * 16 // 128
    loss = _tc_finish(pos, dotp.reshape(r, 128), nv2p.reshape(r, 128),
                      nu2p.reshape(r, 128))
    return loss[0, 0]


# speedup vs baseline: 1.1038x; 1.0145x over previous
import functools

import jax
import jax.numpy as jnp
from jax import lax
from jax.experimental import pallas as pl
from jax.experimental.pallas import tpu as pltpu
from jax.experimental.pallas import tpu_sc as plsc

B = 4096
D = 128
NEG = 5
EPS = 1e-8


def _sc_neg_stats(batch_users, neg_idx_flat, total_user, total_sub):
    info = plsc.get_sparse_core_info()
    NC, NS, L = info.num_cores, info.num_subcores, info.num_lanes
    NW = NC * NS
    bpw = B // NW
    vpw = bpw * NEG
    K = D // L

    mesh = plsc.VectorSubcoreMesh(core_axis_name="c", subcore_axis_name="s")

    @functools.partial(
        pl.kernel,
        mesh=mesh,
        compiler_params=pltpu.CompilerParams(needs_layout_passes=False,
                                             use_tc_tiling_on_sc=False),
        out_type=[
            jax.ShapeDtypeStruct((B * NEG, L), jnp.float32),
            jax.ShapeDtypeStruct((B * NEG, L), jnp.float32),
            jax.ShapeDtypeStruct((B * NEG, L), jnp.float32),
        ],
        scratch_types=[
            pltpu.VMEM((bpw,), jnp.int32),
            pltpu.VMEM((vpw,), jnp.int32),
            pltpu.VMEM((bpw // 8, D), jnp.float32),
            pltpu.VMEM((bpw // 8, D), jnp.float32),
            pltpu.VMEM((bpw // 8, D), jnp.float32),
            pltpu.VMEM((vpw // 8, D), jnp.float32),
            pltpu.VMEM((vpw // 8, D), jnp.float32),
            pltpu.VMEM((vpw // 8, D), jnp.float32),
            pltpu.VMEM((vpw, L), jnp.float32),
            pltpu.VMEM((vpw, L), jnp.float32),
            pltpu.VMEM((vpw, L), jnp.float32),
            pltpu.SemaphoreType.DMA,
            pltpu.SemaphoreType.DMA,
            pltpu.SemaphoreType.DMA,
        ],
    )
    def k(bu_hbm, ni_hbm, tu_hbm, ts_hbm, dot_out, nv2_out, nu2_out,
          uidx_v, nidx_v, u_buf0, u_buf1, u_buf2, v_buf0, v_buf1, v_buf2,
          rdot, rnv2, rnu2, sem0, sem1, sem2):
        u_bufs = (u_buf0, u_buf1, u_buf2)
        v_bufs = (v_buf0, v_buf1, v_buf2)
        wid = lax.axis_index("s") * NC + lax.axis_index("c")
        base = wid * bpw
        vbase = wid * vpw
        P = 8
        rp = bpw // P
        vp = vpw // P
        sems = (sem0, sem1, sem2)
        NB = 3

        pltpu.sync_copy(bu_hbm.at[pl.ds(base, bpw)], uidx_v)
        pltpu.sync_copy(ni_hbm.at[pl.ds(vbase, vpw)], nidx_v)

        def fire(p):
            b = p % NB
            return [
                pltpu.async_copy(tu_hbm.at[uidx_v.at[pl.ds(p * rp, rp)]],
                                 u_bufs[b], sems[b]),
                pltpu.async_copy(ts_hbm.at[nidx_v.at[pl.ds(p * vp, vp)]],
                                 v_bufs[b], sems[b]),
            ]

        pending = {0: fire(0), 1: fire(1)}
        for p in range(P):
            b = p % NB
            for cp in pending.pop(p):
                cp.wait()
            nxt = p + NB - 1
            if nxt < P:
                pending[nxt] = fire(nxt)

            def one_row(r, b=b, p=p):
                ub, vb = u_bufs[b], v_bufs[b]
                u = [ub[r, pl.ds(kk * L, L)] for kk in range(K)]
                accu = u[0] * u[0]
                for kk in range(1, K):
                    accu = accu + u[kk] * u[kk]
                lr0 = r * NEG
                gr0 = p * vp + r * NEG
                for j in range(NEG):
                    v0 = vb[lr0 + j, pl.ds(0, L)]
                    accd = u[0] * v0
                    accv = v0 * v0
                    for kk in range(1, K):
                        vk = vb[lr0 + j, pl.ds(kk * L, L)]
                        accd = accd + u[kk] * vk
                        accv = accv + vk * vk
                    rdot[gr0 + j, :] = accd
                    rnv2[gr0 + j, :] = accv
                    rnu2[gr0 + j, :] = accu

            def row_body(i, carry):
                one_row(2 * i)
                one_row(2 * i + 1)
                return carry

            lax.fori_loop(0, rp // 2, row_body, 0)

        pltpu.sync_copy(rdot, dot_out.at[pl.ds(vbase, vpw)])
        pltpu.sync_copy(rnv2, nv2_out.at[pl.ds(vbase, vpw)])
        pltpu.sync_copy(rnu2, nu2_out.at[pl.ds(vbase, vpw)])

    return k(batch_users, neg_idx_flat, total_user, total_sub)


def _tc_positive(ue, se):
    def body(ue_ref, se_ref, out_ref):
        u = ue_ref[...]
        s = se_ref[...]
        dot = jnp.sum(u * s, axis=1)
        na = jnp.sqrt(jnp.sum(u * u, axis=1))
        nb = jnp.sqrt(jnp.sum(s * s, axis=1))
        out_ref[0, 0] = jnp.sum(dot / jnp.maximum(na * nb, EPS))

    return pl.pallas_call(
        body,
        out_shape=jax.ShapeDtypeStruct((1, 1), jnp.float32),
        out_specs=pl.BlockSpec(memory_space=pltpu.SMEM),
    )(ue, se)


def _tc_finish(pos_sum, dotp, nv2p, nu2p):
    def body(pos_ref, dotp_ref, nv2p_ref, nu2p_ref, out_ref):
        ri = lax.broadcasted_iota(jnp.int32, (128, 8), 0)
        ci = lax.broadcasted_iota(jnp.int32, (128, 8), 1)
        m = (ri // 16 == ci).astype(jnp.float32)
        d = jnp.dot(dotp_ref[...], m, preferred_element_type=jnp.float32)
        v2 = jnp.dot(nv2p_ref[...], m, preferred_element_type=jnp.float32)
        u2 = jnp.dot(nu2p_ref[...], m, preferred_element_type=jnp.float32)
        c = d / jnp.maximum(jnp.sqrt(u2) * jnp.sqrt(v2), EPS)
        neg_sum = jnp.sum(c)

        out_ref[0, 0] = 1.0 - pos_ref[0, 0] / B + neg_sum / (NEG * B)

    return pl.pallas_call(
        body,
        out_shape=jax.ShapeDtypeStruct((1, 1), jnp.float32),
        out_specs=pl.BlockSpec(memory_space=pltpu.SMEM),
    )(pos_sum, dotp, nv2p, nu2p)


def kernel(user_embeddings, subreddit_embeddings, batch_users,
           batch_subreddits, total_user_embeddings, total_subreddit_embeddings,
           negative_indices):
    del batch_subreddits
    bu = batch_users.astype(jnp.int32)
    ni = negative_indices.astype(jnp.int32).reshape(B * NEG)
    dotp, nv2p, nu2p = _sc_neg_stats(bu, ni, total_user_embeddings,
                                     total_subreddit_embeddings)
    pos = _tc_positive(user_embeddings, subreddit_embeddings)
    r = B * NEG * 16 // 128
    loss = _tc_finish(pos, dotp.reshape(r, 128), nv2p.reshape(r, 128),
                      nu2p.reshape(r, 128))
    return loss[0, 0]
